# baseline (speedup 1.0000x reference)
import functools

import jax
import jax.numpy as jnp
from jax import lax
from jax.experimental import pallas as pl
from jax.experimental.pallas import tpu as pltpu
from jax.experimental.pallas import tpu_sc as plsc

B = 16384
D = 64
HIDDEN = 100
NC = 2
NS = 16
NW = NC * NS
BPW = B // NW
CH = 128
NCH = BPW // CH


def _sc_gather(x1, x2, x3, user_embed, movie_embed, category_embed):
    mesh = plsc.VectorSubcoreMesh(core_axis_name="c", subcore_axis_name="s")

    @functools.partial(
        pl.kernel,
        mesh=mesh,
        compiler_params=pltpu.CompilerParams(use_tc_tiling_on_sc=False),
        out_type=[jax.ShapeDtypeStruct((B, D), jnp.float32)] * 3,
        scratch_types=[
            pltpu.VMEM((NCH, CH), jnp.int32),
            pltpu.VMEM((NCH, CH), jnp.int32),
            pltpu.VMEM((NCH, CH), jnp.int32),
            pltpu.VMEM((BPW, D), jnp.float32),
            pltpu.VMEM((BPW, D), jnp.float32),
            pltpu.VMEM((BPW, D), jnp.float32),
            pltpu.SemaphoreType.DMA,
            pltpu.SemaphoreType.DMA,
            pltpu.SemaphoreType.DMA,
        ],
    )
    def gather_kernel(x1h, x2h, x3h, uh, mh, ch, o1h, o2h, o3h,
                      i1, i2, i3, r1, r2, r3, s1, s2, s3):
        wid = lax.axis_index("s") * NC + lax.axis_index("c")
        base = wid * BPW
        pltpu.sync_copy(x1h.at[wid], i1)
        pltpu.sync_copy(x2h.at[wid], i2)
        pltpu.sync_copy(x3h.at[wid], i3)
        cps = []
        for j in range(NCH):
            cps.append(pltpu.async_copy(uh.at[i1.at[j]], r1.at[pl.ds(j * CH, CH)], s1))
            cps.append(pltpu.async_copy(mh.at[i2.at[j]], r2.at[pl.ds(j * CH, CH)], s2))
            cps.append(pltpu.async_copy(ch.at[i3.at[j]], r3.at[pl.ds(j * CH, CH)], s3))
        for cp in cps:
            cp.wait()
        pltpu.sync_copy(r1, o1h.at[pl.ds(base, BPW)])
        pltpu.sync_copy(r2, o2h.at[pl.ds(base, BPW)])
        pltpu.sync_copy(r3, o3h.at[pl.ds(base, BPW)])

    return gather_kernel(
        x1.reshape(NW, NCH, CH),
        x2.reshape(NW, NCH, CH),
        x3.reshape(NW, NCH, CH),
        user_embed, movie_embed, category_embed,
    )


RB = 2048


def _mlp_kernel(e1r, e2r, e3r, w1r, b1r, w2r, b2r, outr):
    h = jnp.dot(e1r[...], w1r[0:D, :], preferred_element_type=jnp.float32)
    h = h + jnp.dot(e2r[...], w1r[D:2 * D, :], preferred_element_type=jnp.float32)
    h = h + jnp.dot(e3r[...], w1r[2 * D:3 * D, :], preferred_element_type=jnp.float32)
    h = jnp.maximum(h + b1r[...], 0.0)
    o = jnp.dot(h, w2r[...], preferred_element_type=jnp.float32) + b2r[...]
    outr[...] = 1.0 / (1.0 + jnp.exp(-o))


def _mlp(e1, e2, e3, W1, b1, W2, b2):
    grid = (B // RB,)
    return pl.pallas_call(
        _mlp_kernel,
        grid=grid,
        in_specs=[
            pl.BlockSpec((RB, D), lambda i: (i, 0)),
            pl.BlockSpec((RB, D), lambda i: (i, 0)),
            pl.BlockSpec((RB, D), lambda i: (i, 0)),
            pl.BlockSpec((3 * D, HIDDEN), lambda i: (0, 0)),
            pl.BlockSpec((1, HIDDEN), lambda i: (0, 0)),
            pl.BlockSpec((HIDDEN, 1), lambda i: (0, 0)),
            pl.BlockSpec((1, 1), lambda i: (0, 0)),
        ],
        out_specs=pl.BlockSpec((RB, 1), lambda i: (i, 0)),
        out_shape=jax.ShapeDtypeStruct((B, 1), jnp.float32),
    )(e1, e2, e3, W1, b1, W2, b2)


def kernel(x1, x2, x3, user_embed, movie_embed, category_embed, W1, b1, W2, b2):
    x1 = x1.astype(jnp.int32)
    x2 = x2.astype(jnp.int32)
    x3 = x3.astype(jnp.int32)
    e1, e2, e3 = _sc_gather(x1, x2, x3, user_embed, movie_embed, category_embed)
    return _mlp(e1, e2, e3, W1,
                b1.reshape(1, HIDDEN), W2, b2.reshape(1, 1))
